# baseline (speedup 1.0000x reference)
import functools

import jax
import jax.numpy as jnp
from jax import lax
from jax.experimental import pallas as pl
from jax.experimental.pallas import tpu as pltpu
from jax.experimental.pallas import tpu_sc as plsc

S = 200
B = 4096
DW = 64
DP = 64
D = DW + DP
L = 16
NC, NS = 2, 16
NW = NC * NS
BPW = B // NW
NBUF = 4

IDX_ROWS, IDX_COLS = 2, 100

_mesh = plsc.VectorSubcoreMesh(core_axis_name="c", subcore_axis_name="s")


def _body(inpt_hbm, lens_hbm, wtab_hbm, ptab_hbm, out_hbm, mean_hbm,
          lens_v, i0, i1, i2, i3, w0, w1, w2, w3, posm_v, psum_v, mean_v,
          gi0, gi1, gi2, gi3, g0, g1, g2, g3, s0, s1, s2, s3, psem):
    cid = lax.axis_index("c")
    sid = lax.axis_index("s")
    wid = sid * NC + cid
    b0 = wid * BPW

    idxs = [i0, i1, i2, i3]
    isems = [gi0, gi1, gi2, gi3]
    bufs = [w0, w1, w2, w3]
    gsems = [g0, g1, g2, g3]
    wsems = [s0, s1, s2, s3]

    pltpu.sync_copy(lens_hbm.at[pl.ds(b0, BPW)], lens_v)
    pltpu.sync_copy(ptab_hbm, posm_v)

    zeros16 = jnp.zeros((16,), jnp.float32)

    def issue_idx(c, q):
        pltpu.async_copy(inpt_hbm.at[b0 + c], idxs[q], isems[q])

    def wait_idx(c, q):
        pltpu.make_async_copy(inpt_hbm.at[b0 + c], idxs[q],
                              isems[q]).wait()

    def issue_gather(q):
        pltpu.async_copy(wtab_hbm.at[idxs[q].at[0]],
                         bufs[q].at[pl.ds(0, IDX_COLS)], gsems[q])
        pltpu.async_copy(wtab_hbm.at[idxs[q].at[1]],
                         bufs[q].at[pl.ds(IDX_COLS, IDX_COLS)], gsems[q])

    def wait_gather(q):
        pltpu.make_async_copy(wtab_hbm.at[idxs[q].at[0]],
                              bufs[q].at[pl.ds(0, IDX_COLS)],
                              gsems[q]).wait()
        pltpu.make_async_copy(wtab_hbm.at[idxs[q].at[1]],
                              bufs[q].at[pl.ds(IDX_COLS, IDX_COLS)],
                              gsems[q]).wait()

    def wait_write(c, q):
        pltpu.make_async_copy(bufs[q], out_hbm.at[:, b0 + c, pl.ds(0, DW)],
                              wsems[q]).wait()

    def pos_zero_and_issue(c_new, len_new, len_prev):
        def pz_body(s, _):
            for k in range(DP // L):
                posm_v[s, pl.ds(k * L, L)] = zeros16
            return 0
        lax.fori_loop(len_new, len_prev, pz_body, 0)
        pltpu.async_copy(posm_v, out_hbm.at[:, b0 + c_new, pl.ds(DW, DP)],
                         psem)

    def wait_pos(c):
        pltpu.make_async_copy(posm_v, out_hbm.at[:, b0 + c, pl.ds(DW, DP)],
                              psem).wait()

    def psum_body(s, acc):
        nacc = []
        for k in range(DP // L):
            a = acc[k] + posm_v[s, pl.ds(k * L, L)]
            psum_v[s, pl.ds(k * L, L)] = a
            nacc.append(a)
        return tuple(nacc)
    lax.fori_loop(0, S, psum_body, (zeros16,) * (DP // L))

    for q in range(NBUF):
        issue_idx(q, q)
    wait_idx(0, 0)
    issue_gather(0)
    wait_idx(1, 1)
    issue_gather(1)
    len0 = lens_v[pl.ds(0, 1)][0]
    pos_zero_and_issue(0, len0, S)

    def slot(c, q):
        b = b0 + c
        len_c = lens_v[pl.ds(c, 1)][0]

        qn = (q + 2) % NBUF

        @pl.when(c >= 2)
        def _():
            wait_write(c - 2, qn)

        @pl.when(c + 2 < BPW)
        def _():
            wait_idx(c + 2, qn)
            issue_gather(qn)

        wait_gather(q)
        word_v = bufs[q]

        @pl.when(c + 4 < BPW)
        def _():
            issue_idx(c + 4, q)

        def acc_body(s, acc):
            return tuple(acc[k] + word_v[s, pl.ds(k * L, L)]
                         for k in range(DW // L))
        acc = lax.fori_loop(0, len_c, acc_body, (zeros16,) * (DW // L))

        def ztail_body(s, _):
            for k in range(DW // L):
                word_v[s, pl.ds(k * L, L)] = zeros16
            return 0
        lax.fori_loop(len_c, S, ztail_body, 0)

        pltpu.async_copy(word_v, out_hbm.at[:, b, pl.ds(0, DW)], wsems[q])

        inv = jnp.full((16,), 1.0, jnp.float32) / len_c.astype(jnp.float32)
        for k in range(DW // L):
            mean_v[c, pl.ds(k * L, L)] = acc[k] * inv
        for k in range(DP // L):
            mean_v[c, pl.ds(DW + k * L, L)] = (
                psum_v[len_c - 1, pl.ds(k * L, L)] * inv)

        wait_pos(c)

        @pl.when(c + 1 < BPW)
        def _():
            len_n = lens_v[pl.ds(c + 1, 1)][0]
            pos_zero_and_issue(c + 1, len_n, len_c)

    def body(it, _):
        j = it * NBUF
        for q in range(NBUF):
            slot(j + q, q)
        return 0

    lax.fori_loop(0, BPW // NBUF, body, 0)

    wait_write(BPW - 2, (BPW - 2) % NBUF)
    wait_write(BPW - 1, (BPW - 1) % NBUF)
    pltpu.sync_copy(mean_v, mean_hbm.at[pl.ds(b0, BPW), :])


_encoder = functools.partial(
    pl.kernel,
    out_type=(jax.ShapeDtypeStruct((S, B, D), jnp.float32),
              jax.ShapeDtypeStruct((B, D), jnp.float32)),
    mesh=_mesh,
    compiler_params=pltpu.CompilerParams(use_tc_tiling_on_sc=False),
    scratch_types=[
        pltpu.VMEM((BPW,), jnp.int32),
        pltpu.VMEM((IDX_ROWS, IDX_COLS), jnp.int32),
        pltpu.VMEM((IDX_ROWS, IDX_COLS), jnp.int32),
        pltpu.VMEM((IDX_ROWS, IDX_COLS), jnp.int32),
        pltpu.VMEM((IDX_ROWS, IDX_COLS), jnp.int32),
        pltpu.VMEM((S, DW), jnp.float32),
        pltpu.VMEM((S, DW), jnp.float32),
        pltpu.VMEM((S, DW), jnp.float32),
        pltpu.VMEM((S, DW), jnp.float32),
        pltpu.VMEM((S, DP), jnp.float32),
        pltpu.VMEM((S, DP), jnp.float32),
        pltpu.VMEM((BPW, D), jnp.float32),
        pltpu.SemaphoreType.DMA,
        pltpu.SemaphoreType.DMA,
        pltpu.SemaphoreType.DMA,
        pltpu.SemaphoreType.DMA,
        pltpu.SemaphoreType.DMA,
        pltpu.SemaphoreType.DMA,
        pltpu.SemaphoreType.DMA,
        pltpu.SemaphoreType.DMA,
        pltpu.SemaphoreType.DMA,
        pltpu.SemaphoreType.DMA,
        pltpu.SemaphoreType.DMA,
        pltpu.SemaphoreType.DMA,
        pltpu.SemaphoreType.DMA,
    ],
)(_body)


def kernel(input, input_lens, word_table, pos_table):
    inpt = input.astype(jnp.int32).T.reshape(B, IDX_ROWS, IDX_COLS)
    return _encoder(inpt, input_lens.astype(jnp.int32),
                    word_table, pos_table)


# speedup vs baseline: 1.0009x; 1.0009x over previous
import functools

import jax
import jax.numpy as jnp
from jax import lax
from jax.experimental import pallas as pl
from jax.experimental.pallas import tpu as pltpu
from jax.experimental.pallas import tpu_sc as plsc

S = 200
B = 4096
DW = 64
DP = 64
D = DW + DP
L = 16
NC, NS = 2, 16
NW = NC * NS
BPW = B // NW
NBUF = 4

IDX_ROWS, IDX_COLS = 2, 100

_mesh = plsc.VectorSubcoreMesh(core_axis_name="c", subcore_axis_name="s")


def _body(inpt_hbm, lens_hbm, wtab_hbm, ptab_hbm, out_hbm, mean_hbm,
          lens_v, i0, i1, i2, i3, w0, w1, w2, w3, posm_v, psum_v, mean_v,
          gi0, gi1, gi2, gi3, g0, g1, g2, g3, s0, s1, s2, s3, psem):
    cid = lax.axis_index("c")
    sid = lax.axis_index("s")
    wid = sid * NC + cid
    b0 = wid * BPW

    idxs = [i0, i1, i2, i3]
    isems = [gi0, gi1, gi2, gi3]
    bufs = [w0, w1, w2, w3]
    gsems = [g0, g1, g2, g3]
    wsems = [s0, s1, s2, s3]

    pltpu.sync_copy(lens_hbm.at[pl.ds(b0, BPW)], lens_v)
    pltpu.sync_copy(ptab_hbm, posm_v)

    zeros16 = jnp.zeros((16,), jnp.float32)

    def issue_idx(c, q):
        pltpu.async_copy(inpt_hbm.at[b0 + c], idxs[q], isems[q])

    def wait_idx(c, q):
        pltpu.make_async_copy(inpt_hbm.at[b0 + c], idxs[q],
                              isems[q]).wait()

    def issue_gather(q):
        pltpu.async_copy(wtab_hbm.at[idxs[q].at[0]],
                         bufs[q].at[pl.ds(0, IDX_COLS)], gsems[q])
        pltpu.async_copy(wtab_hbm.at[idxs[q].at[1]],
                         bufs[q].at[pl.ds(IDX_COLS, IDX_COLS)], gsems[q])

    def wait_gather(q):
        pltpu.make_async_copy(wtab_hbm.at[idxs[q].at[0]],
                              bufs[q].at[pl.ds(0, IDX_COLS)],
                              gsems[q]).wait()
        pltpu.make_async_copy(wtab_hbm.at[idxs[q].at[1]],
                              bufs[q].at[pl.ds(IDX_COLS, IDX_COLS)],
                              gsems[q]).wait()

    def wait_write(c, q):
        pltpu.make_async_copy(bufs[q], out_hbm.at[:, b0 + c, pl.ds(0, DW)],
                              wsems[q]).wait()

    def pos_zero_and_issue(c_new, len_new, len_prev):
        def pz_body(s, _):
            for k in range(DP // L):
                posm_v[s, pl.ds(k * L, L)] = zeros16
            return 0
        lax.fori_loop(len_new, len_prev, pz_body, 0)
        pltpu.async_copy(posm_v, out_hbm.at[:, b0 + c_new, pl.ds(DW, DP)],
                         psem)

    def wait_pos(c):
        pltpu.make_async_copy(posm_v, out_hbm.at[:, b0 + c, pl.ds(DW, DP)],
                              psem).wait()

    def psum_body(s, acc):
        nacc = []
        for k in range(DP // L):
            a = acc[k] + posm_v[s, pl.ds(k * L, L)]
            psum_v[s, pl.ds(k * L, L)] = a
            nacc.append(a)
        return tuple(nacc)
    lax.fori_loop(0, S, psum_body, (zeros16,) * (DP // L))

    for q in range(NBUF):
        issue_idx(q, q)
    wait_idx(0, 0)
    issue_gather(0)
    wait_idx(1, 1)
    issue_gather(1)
    len0 = lens_v[pl.ds(0, 1)][0]
    pos_zero_and_issue(0, len0, S)

    def slot(c, q):
        b = b0 + c
        len_c = lens_v[pl.ds(c, 1)][0]

        qn = (q + 2) % NBUF

        @pl.when(c >= 2)
        def _():
            wait_write(c - 2, qn)

        @pl.when(c + 2 < BPW)
        def _():
            wait_idx(c + 2, qn)
            issue_gather(qn)

        wait_gather(q)
        word_v = bufs[q]

        @pl.when(c + 4 < BPW)
        def _():
            issue_idx(c + 4, q)

        def acc_body(s, acc):
            return tuple(acc[k] + word_v[s, pl.ds(k * L, L)]
                         for k in range(DW // L))
        acc = lax.fori_loop(0, len_c, acc_body, (zeros16,) * (DW // L))

        def ztail_body(s, _):
            for k in range(DW // L):
                word_v[s, pl.ds(k * L, L)] = zeros16
            return 0
        lax.fori_loop(len_c, S, ztail_body, 0)

        pltpu.async_copy(word_v, out_hbm.at[:, b, pl.ds(0, DW)], wsems[q])

        inv = jnp.full((16,), 1.0, jnp.float32) / len_c.astype(jnp.float32)
        for k in range(DW // L):
            mean_v[c, pl.ds(k * L, L)] = acc[k] * inv
        for k in range(DP // L):
            mean_v[c, pl.ds(DW + k * L, L)] = (
                psum_v[len_c - 1, pl.ds(k * L, L)] * inv)

        wait_pos(c)

        @pl.when(c + 1 < BPW)
        def _():
            len_n = lens_v[pl.ds(c + 1, 1)][0]
            pos_zero_and_issue(c + 1, len_n, len_c)

    def body(it, _):
        j = it * NBUF
        for q in range(NBUF):
            slot(j + q, q)
        return 0

    lax.fori_loop(0, BPW // NBUF, body, 0)

    wait_write(BPW - 2, (BPW - 2) % NBUF)
    wait_write(BPW - 1, (BPW - 1) % NBUF)
    pltpu.sync_copy(mean_v, mean_hbm.at[pl.ds(b0, BPW), :])


_encoder = functools.partial(
    pl.kernel,
    out_type=(jax.ShapeDtypeStruct((S, B, D), jnp.float32),
              jax.ShapeDtypeStruct((B, D), jnp.float32)),
    mesh=_mesh,
    compiler_params=pltpu.CompilerParams(use_tc_tiling_on_sc=False),
    scratch_types=[
        pltpu.VMEM((BPW,), jnp.int32),
        pltpu.VMEM((IDX_ROWS, IDX_COLS), jnp.int32),
        pltpu.VMEM((IDX_ROWS, IDX_COLS), jnp.int32),
        pltpu.VMEM((IDX_ROWS, IDX_COLS), jnp.int32),
        pltpu.VMEM((IDX_ROWS, IDX_COLS), jnp.int32),
        pltpu.VMEM((S, DW), jnp.float32),
        pltpu.VMEM((S, DW), jnp.float32),
        pltpu.VMEM((S, DW), jnp.float32),
        pltpu.VMEM((S, DW), jnp.float32),
        pltpu.VMEM((S, DP), jnp.float32),
        pltpu.VMEM((S, DP), jnp.float32),
        pltpu.VMEM((BPW, D), jnp.float32),
        pltpu.SemaphoreType.DMA,
        pltpu.SemaphoreType.DMA,
        pltpu.SemaphoreType.DMA,
        pltpu.SemaphoreType.DMA,
        pltpu.SemaphoreType.DMA,
        pltpu.SemaphoreType.DMA,
        pltpu.SemaphoreType.DMA,
        pltpu.SemaphoreType.DMA,
        pltpu.SemaphoreType.DMA,
        pltpu.SemaphoreType.DMA,
        pltpu.SemaphoreType.DMA,
        pltpu.SemaphoreType.DMA,
        pltpu.SemaphoreType.DMA,
    ],
)(_body)


TRB = 128


def _tr_body(x_ref, o_ref):
    o_ref[...] = x_ref[...].T


_transpose = pl.pallas_call(
    _tr_body,
    grid=(B // TRB,),
    in_specs=[pl.BlockSpec((S, TRB), lambda i: (0, i))],
    out_specs=pl.BlockSpec((TRB, S), lambda i: (i, 0)),
    out_shape=jax.ShapeDtypeStruct((B, S), jnp.int32),
)


def kernel(input, input_lens, word_table, pos_table):
    inpt = _transpose(input.astype(jnp.int32)).reshape(B, IDX_ROWS, IDX_COLS)
    return _encoder(inpt, input_lens.astype(jnp.int32),
                    word_table, pos_table)


# speedup vs baseline: 1.0053x; 1.0044x over previous
import functools

import jax
import jax.numpy as jnp
from jax import lax
from jax.experimental import pallas as pl
from jax.experimental.pallas import tpu as pltpu
from jax.experimental.pallas import tpu_sc as plsc

S = 200
B = 4096
DW = 64
DP = 64
D = DW + DP
L = 16
NC, NS = 2, 16
NW = NC * NS
BPW = B // NW
NBUF = 4

GA, GB = 104, 96

_mesh = plsc.VectorSubcoreMesh(core_axis_name="c", subcore_axis_name="s")


def _body(inpt_hbm, lens_hbm, wtab_hbm, ptab_hbm, out_hbm, mean_hbm,
          lens_v, i0, i1, i2, i3, w0, w1, w2, w3, posm_v, psum_v, mean_v,
          gi0, gi1, gi2, gi3, g0, g1, g2, g3, s0, s1, s2, s3, psem):
    cid = lax.axis_index("c")
    sid = lax.axis_index("s")
    wid = sid * NC + cid
    b0 = wid * BPW

    idxs = [i0, i1, i2, i3]
    isems = [gi0, gi1, gi2, gi3]
    bufs = [w0, w1, w2, w3]
    gsems = [g0, g1, g2, g3]
    wsems = [s0, s1, s2, s3]

    pltpu.sync_copy(lens_hbm.at[pl.ds(b0, BPW)], lens_v)
    pltpu.sync_copy(ptab_hbm, posm_v)

    zeros16 = jnp.zeros((16,), jnp.float32)

    def issue_idx(c, q):
        pltpu.async_copy(inpt_hbm.at[b0 + c], idxs[q], isems[q])

    def wait_idx(c, q):
        pltpu.make_async_copy(inpt_hbm.at[b0 + c], idxs[q],
                              isems[q]).wait()

    def issue_gather(q):
        pltpu.async_copy(wtab_hbm.at[idxs[q].at[pl.ds(0, GA)]],
                         bufs[q].at[pl.ds(0, GA)], gsems[q])
        pltpu.async_copy(wtab_hbm.at[idxs[q].at[pl.ds(GA, GB)]],
                         bufs[q].at[pl.ds(GA, GB)], gsems[q])

    def wait_gather(q):
        pltpu.make_async_copy(wtab_hbm.at[idxs[q].at[pl.ds(0, GA)]],
                              bufs[q].at[pl.ds(0, GA)],
                              gsems[q]).wait()
        pltpu.make_async_copy(wtab_hbm.at[idxs[q].at[pl.ds(GA, GB)]],
                              bufs[q].at[pl.ds(GA, GB)],
                              gsems[q]).wait()

    def wait_write(c, q):
        pltpu.make_async_copy(bufs[q], out_hbm.at[:, b0 + c, pl.ds(0, DW)],
                              wsems[q]).wait()

    def pos_zero_and_issue(c_new, len_new, len_prev):
        def pz_body(s, _):
            for k in range(DP // L):
                posm_v[s, pl.ds(k * L, L)] = zeros16
            return 0
        lax.fori_loop(len_new, len_prev, pz_body, 0)
        pltpu.async_copy(posm_v, out_hbm.at[:, b0 + c_new, pl.ds(DW, DP)],
                         psem)

    def wait_pos(c):
        pltpu.make_async_copy(posm_v, out_hbm.at[:, b0 + c, pl.ds(DW, DP)],
                              psem).wait()

    def psum_body(s, acc):
        nacc = []
        for k in range(DP // L):
            a = acc[k] + posm_v[s, pl.ds(k * L, L)]
            psum_v[s, pl.ds(k * L, L)] = a
            nacc.append(a)
        return tuple(nacc)
    lax.fori_loop(0, S, psum_body, (zeros16,) * (DP // L))

    for q in range(NBUF):
        issue_idx(q, q)
    wait_idx(0, 0)
    issue_gather(0)
    wait_idx(1, 1)
    issue_gather(1)
    len0 = lens_v[pl.ds(0, 1)][0]
    pos_zero_and_issue(0, len0, S)

    def slot(c, q):
        b = b0 + c
        len_c = lens_v[pl.ds(c, 1)][0]

        qn = (q + 2) % NBUF

        @pl.when(c >= 2)
        def _():
            wait_write(c - 2, qn)

        @pl.when(c + 2 < BPW)
        def _():
            wait_idx(c + 2, qn)
            issue_gather(qn)

        wait_gather(q)
        word_v = bufs[q]

        @pl.when(c + 4 < BPW)
        def _():
            issue_idx(c + 4, q)

        def acc_body(s, acc):
            return tuple(acc[k] + word_v[s, pl.ds(k * L, L)]
                         for k in range(DW // L))
        acc = lax.fori_loop(0, len_c, acc_body, (zeros16,) * (DW // L))

        def ztail_body(s, _):
            for k in range(DW // L):
                word_v[s, pl.ds(k * L, L)] = zeros16
            return 0
        lax.fori_loop(len_c, S, ztail_body, 0)

        pltpu.async_copy(word_v, out_hbm.at[:, b, pl.ds(0, DW)], wsems[q])

        inv = jnp.full((16,), 1.0, jnp.float32) / len_c.astype(jnp.float32)
        for k in range(DW // L):
            mean_v[c, pl.ds(k * L, L)] = acc[k] * inv
        for k in range(DP // L):
            mean_v[c, pl.ds(DW + k * L, L)] = (
                psum_v[len_c - 1, pl.ds(k * L, L)] * inv)

        wait_pos(c)

        @pl.when(c + 1 < BPW)
        def _():
            len_n = lens_v[pl.ds(c + 1, 1)][0]
            pos_zero_and_issue(c + 1, len_n, len_c)

    def body(it, _):
        j = it * NBUF
        for q in range(NBUF):
            slot(j + q, q)
        return 0

    lax.fori_loop(0, BPW // NBUF, body, 0)

    wait_write(BPW - 2, (BPW - 2) % NBUF)
    wait_write(BPW - 1, (BPW - 1) % NBUF)
    pltpu.sync_copy(mean_v, mean_hbm.at[pl.ds(b0, BPW), :])


_encoder = functools.partial(
    pl.kernel,
    out_type=(jax.ShapeDtypeStruct((S, B, D), jnp.float32),
              jax.ShapeDtypeStruct((B, D), jnp.float32)),
    mesh=_mesh,
    compiler_params=pltpu.CompilerParams(use_tc_tiling_on_sc=False),
    scratch_types=[
        pltpu.VMEM((BPW,), jnp.int32),
        pltpu.VMEM((S,), jnp.int32),
        pltpu.VMEM((S,), jnp.int32),
        pltpu.VMEM((S,), jnp.int32),
        pltpu.VMEM((S,), jnp.int32),
        pltpu.VMEM((S, DW), jnp.float32),
        pltpu.VMEM((S, DW), jnp.float32),
        pltpu.VMEM((S, DW), jnp.float32),
        pltpu.VMEM((S, DW), jnp.float32),
        pltpu.VMEM((S, DP), jnp.float32),
        pltpu.VMEM((S, DP), jnp.float32),
        pltpu.VMEM((BPW, D), jnp.float32),
        pltpu.SemaphoreType.DMA,
        pltpu.SemaphoreType.DMA,
        pltpu.SemaphoreType.DMA,
        pltpu.SemaphoreType.DMA,
        pltpu.SemaphoreType.DMA,
        pltpu.SemaphoreType.DMA,
        pltpu.SemaphoreType.DMA,
        pltpu.SemaphoreType.DMA,
        pltpu.SemaphoreType.DMA,
        pltpu.SemaphoreType.DMA,
        pltpu.SemaphoreType.DMA,
        pltpu.SemaphoreType.DMA,
        pltpu.SemaphoreType.DMA,
    ],
)(_body)


TRB = 128


def _tr_body(x_ref, o_ref):
    o_ref[...] = x_ref[...].T


_transpose = pl.pallas_call(
    _tr_body,
    grid=(B // TRB,),
    in_specs=[pl.BlockSpec((S, TRB), lambda i: (0, i))],
    out_specs=pl.BlockSpec((TRB, S), lambda i: (i, 0)),
    out_shape=jax.ShapeDtypeStruct((B, S), jnp.int32),
)


def kernel(input, input_lens, word_table, pos_table):
    inpt = _transpose(input.astype(jnp.int32))
    return _encoder(inpt, input_lens.astype(jnp.int32),
                    word_table, pos_table)
